# baseline (speedup 1.0000x reference)
import functools

import jax
import jax.numpy as jnp
from jax import lax
from jax.experimental import pallas as pl
from jax.experimental.pallas import tpu as pltpu
from jax.experimental.pallas import tpu_sc as plsc

_EMBED = 64
_LANES = 16
_NC = 2
_NS = 16
_NW = _NC * _NS
_SEQ = 50
_SEQP = 56
_BPG = 2
_G = _BPG * _SEQP
_SLOTS = 4
_AHEAD = 3
_VPR = _EMBED // _LANES


def _emb_body(n_groups, table_hbm, idx_hbm, out_hbm, idx_v, bufg, bufo, sg, ss):
    wid = lax.axis_index("s") * _NC + lax.axis_index("c")
    base = wid * n_groups * _G

    pltpu.sync_copy(idx_hbm.at[wid], idx_v)

    ones_i = jnp.ones((_LANES,), jnp.int32)
    zeros_i = jnp.zeros((_LANES,), jnp.int32)
    zeros_f = jnp.zeros((_LANES,), jnp.float32)

    def fire_gather(g):
        slot = lax.rem(g, _SLOTS)
        pltpu.async_copy(
            table_hbm.at[idx_v.at[g]],
            bufg.at[pl.ds(slot * _G, _G)],
            sg,
        )

    def wait_gather(g):
        slot = lax.rem(g, _SLOTS)
        pltpu.make_async_copy(
            table_hbm.at[idx_v.at[g]],
            bufg.at[pl.ds(slot * _G, _G)],
            sg,
        ).wait()

    def fire_scatter(g):
        slot = lax.rem(g, _SLOTS)
        pltpu.async_copy(
            bufo.at[pl.ds(slot * _G, _G)],
            out_hbm.at[pl.ds(base + g * _G, _G)],
            ss,
        )

    def wait_scatter_one():
        pltpu.make_async_copy(
            bufo.at[pl.ds(0, _G)],
            out_hbm.at[pl.ds(base, _G)],
            ss,
        ).wait()

    def fixup(g):
        slot = lax.rem(g, _SLOTS)
        macc = zeros_i
        for k in range(_G // _LANES):
            v = idx_v[g, pl.ds(k * _LANES, _LANES)]
            macc = macc | jnp.where(v == 0, ones_i, zeros_i)
        any_zero = macc[0]
        for l in range(1, _LANES):
            any_zero = any_zero | macc[l]

        @pl.when(any_zero != 0)
        def _():
            for k in range(_G // _LANES):
                v = idx_v[g, pl.ds(k * _LANES, _LANES)]
                for l in range(_LANES):
                    s = v[l]

                    @pl.when(s == 0)
                    def _zero_row():
                        r = slot * _G + k * _LANES + l
                        for c in range(_VPR):
                            bufg[r, pl.ds(c * _LANES, _LANES)] = zeros_f

    def widen(g):
        slot = lax.rem(g, _SLOTS)

        def wrow(r, carry):
            rr = slot * _G + r
            for c in range(_VPR):
                bufo[rr, pl.ds(c * _LANES, _LANES)] = (
                    bufg[rr, pl.ds(c * _LANES, _LANES)])
            return carry

        lax.fori_loop(0, _G, wrow, 0)

    for g in range(_AHEAD):
        fire_gather(g)

    def step(g, carry):
        wait_gather(g)
        fixup(g)

        @pl.when(g >= _SLOTS)
        def _():
            wait_scatter_one()

        widen(g)
        fire_scatter(g)

        @pl.when(g + _AHEAD < n_groups)
        def _():
            fire_gather(g + _AHEAD)

        return carry

    lax.fori_loop(0, n_groups, step, 0)
    n_waited = max(0, n_groups - _SLOTS)
    for _ in range(n_groups - n_waited):
        wait_scatter_one()


@functools.lru_cache(maxsize=None)
def _make_emb(vocab, n_groups):
    b = _NW * n_groups * _BPG
    mesh = plsc.VectorSubcoreMesh(core_axis_name="c", subcore_axis_name="s")
    return pl.kernel(
        functools.partial(_emb_body, n_groups),
        mesh=mesh,
        compiler_params=pltpu.CompilerParams(use_tc_tiling_on_sc=False),
        out_type=jax.ShapeDtypeStruct((b * _SEQP, 2 * _EMBED), jnp.float32),
        scratch_types=[
            pltpu.VMEM((n_groups, _G), jnp.int32),
            pltpu.VMEM((_SLOTS * _G, _EMBED), jnp.float32),
            pltpu.VMEM((_SLOTS * _G, 2 * _EMBED), jnp.float32),
            pltpu.SemaphoreType.DMA,
            pltpu.SemaphoreType.DMA,
        ],
    )


def kernel(indices, table):
    b, s = indices.shape
    n_groups = b // (_NW * _BPG)
    idx = jnp.pad(indices.astype(jnp.int32), ((0, 0), (0, _SEQP - s)),
                  constant_values=1)
    idx = idx.reshape(_NW, n_groups, _G)
    out = _make_emb(table.shape[0], n_groups)(table, idx)
    return out.reshape(b, _SEQP, 2 * _EMBED)[:, :s, :_EMBED]


# speedup vs baseline: 2.7676x; 2.7676x over previous
import functools

import jax
import jax.numpy as jnp
from jax import lax
from jax.experimental import pallas as pl
from jax.experimental.pallas import tpu as pltpu
from jax.experimental.pallas import tpu_sc as plsc

_EMBED = 64
_LANES = 16
_NC = 2
_NS = 16
_NW = _NC * _NS
_R = 128
_SLOTS = 8
_AHEAD = 6


def _emb_body(n_groups, table_hbm, idx_hbm, out_hbm, idx_v, buf, sg, ss):
    per_w = n_groups * _R
    wid = lax.axis_index("s") * _NC + lax.axis_index("c")
    base = wid * per_w

    pltpu.sync_copy(idx_hbm.at[wid], idx_v)

    ones_i = jnp.ones((_LANES,), jnp.int32)
    zeros_i = jnp.zeros((_LANES,), jnp.int32)
    zeros_f = jnp.zeros((_LANES,), jnp.float32)

    def fire_gather(g):
        slot = lax.rem(g, _SLOTS)
        pltpu.async_copy(
            table_hbm.at[idx_v.at[g]],
            buf.at[pl.ds(slot * _R, _R)],
            sg,
        )

    def wait_gather(g):
        slot = lax.rem(g, _SLOTS)
        pltpu.make_async_copy(
            table_hbm.at[idx_v.at[g]],
            buf.at[pl.ds(slot * _R, _R)],
            sg,
        ).wait()

    def fire_scatter(g):
        slot = lax.rem(g, _SLOTS)
        pltpu.async_copy(
            buf.at[pl.ds(slot * _R, _R)],
            out_hbm.at[pl.ds(base + g * _R, _R)],
            ss,
        )

    def wait_scatter_one():
        pltpu.make_async_copy(
            buf.at[pl.ds(0, _R)],
            out_hbm.at[pl.ds(base, _R)],
            ss,
        ).wait()

    def fixup(g):
        slot = lax.rem(g, _SLOTS)
        macc = zeros_i
        for k in range(_R // _LANES):
            v = idx_v[g, pl.ds(k * _LANES, _LANES)]
            macc = macc | jnp.where(v == 0, ones_i, zeros_i)
        any_zero = macc[0]
        for l in range(1, _LANES):
            any_zero = any_zero | macc[l]

        @pl.when(any_zero != 0)
        def _():
            def body(k, carry):
                v = idx_v[g, pl.ds(k * _LANES, _LANES)]
                for l in range(_LANES):
                    s = v[l]

                    @pl.when(s == 0)
                    def _zero_row():
                        r = slot * _R + k * _LANES + l
                        for c in range(_EMBED // _LANES):
                            buf[r, pl.ds(c * _LANES, _LANES)] = zeros_f

                return carry

            lax.fori_loop(0, _R // _LANES, body, 0)

    for g in range(_AHEAD):
        fire_gather(g)

    def step(g, carry):
        wait_gather(g)
        fixup(g)
        fire_scatter(g)

        @pl.when(g + _AHEAD < n_groups)
        def _():
            @pl.when(g >= _SLOTS - _AHEAD)
            def _():
                wait_scatter_one()

            fire_gather(g + _AHEAD)

        return carry

    lax.fori_loop(0, n_groups, step, 0)
    n_waited = max(0, (n_groups - _AHEAD) - (_SLOTS - _AHEAD))
    for _ in range(n_groups - n_waited):
        wait_scatter_one()


@functools.lru_cache(maxsize=None)
def _make_emb(vocab, n_groups):
    n = _NW * n_groups * _R
    mesh = plsc.VectorSubcoreMesh(core_axis_name="c", subcore_axis_name="s")
    return pl.kernel(
        functools.partial(_emb_body, n_groups),
        mesh=mesh,
        compiler_params=pltpu.CompilerParams(use_tc_tiling_on_sc=False),
        out_type=jax.ShapeDtypeStruct((n, _EMBED), jnp.float32),
        scratch_types=[
            pltpu.VMEM((n_groups, _R), jnp.int32),
            pltpu.VMEM((_SLOTS * _R, _EMBED), jnp.float32),
            pltpu.SemaphoreType.DMA,
            pltpu.SemaphoreType.DMA,
        ],
    )


def kernel(indices, table):
    b, s = indices.shape
    n = b * s
    n_groups = n // (_NW * _R)
    idx = indices.reshape(_NW, n_groups, _R).astype(jnp.int32)
    out = _make_emb(table.shape[0], n_groups)(table, idx)
    return out.reshape(b, s, _EMBED)
